# baseline (speedup 1.0000x reference)
import functools

import jax
import jax.numpy as jnp
from jax import lax
from jax.experimental import pallas as pl
from jax.experimental.pallas import tpu as pltpu
from jax.experimental.pallas import tpu_sc as plsc

NC = 2
NS = 16
L = 16
NW = NC * NS

BATCH = 16384
D = 32
BPW = BATCH // NW
CHUNK = 128
NCHUNK = BPW // CHUNK
GROUPS_PER_CHUNK = CHUNK // L
NGROUP = BPW // L


def _mf_body(x_hbm, uemb_hbm, iemb_hbm, out_hbm,
             xv, uidx, iidx, urows, irows, outv, sem):
    wid = lax.axis_index("s") * NC + lax.axis_index("c")
    base = wid * BPW

    pltpu.sync_copy(x_hbm.at[pl.ds(base, BPW), :], xv)

    iota = lax.broadcasted_iota(jnp.int32, (L,), 0)
    zero = jnp.zeros((L,), jnp.int32)
    one = jnp.ones((L,), jnp.int32)

    copies = []
    for c in range(NCHUNK):
        def split_body(g, _, c=c):
            rows = (c * GROUPS_PER_CHUNK + g) * L + iota
            u = plsc.load_gather(xv, [rows, zero])
            v = plsc.load_gather(xv, [rows, one])
            uidx[c, pl.ds(g * L, L)] = u
            iidx[c, pl.ds(g * L, L)] = v
            return 0

        lax.fori_loop(0, GROUPS_PER_CHUNK, split_body, 0, unroll=True)
        copies.append(pltpu.async_copy(
            uemb_hbm.at[uidx.at[c]],
            urows.at[pl.ds(c * CHUNK, CHUNK), :], sem))
        copies.append(pltpu.async_copy(
            iemb_hbm.at[iidx.at[c]],
            irows.at[pl.ds(c * CHUNK, CHUNK), :], sem))
    for cp in copies:
        cp.wait()

    def group_body(g, _):
        rows = g * L + iota
        acc = jnp.zeros((L,), jnp.float32)
        for d in range(D):
            dvec = jnp.full((L,), d, jnp.int32)
            u = plsc.load_gather(urows, [rows, dvec])
            v = plsc.load_gather(irows, [rows, dvec])
            acc = acc + u * v
        outv[pl.ds(g * L, L)] = acc
        return 0

    lax.fori_loop(0, NGROUP, group_body, 0)

    pltpu.sync_copy(outv, out_hbm.at[pl.ds(base, BPW)])


@jax.jit
def kernel(x, user_embedding, item_embedding):
    mesh = plsc.VectorSubcoreMesh(core_axis_name="c", subcore_axis_name="s")
    run = pl.kernel(
        _mf_body,
        out_type=jax.ShapeDtypeStruct((BATCH,), jnp.float32),
        mesh=mesh,
        compiler_params=pltpu.CompilerParams(
            needs_layout_passes=False, use_tc_tiling_on_sc=False),
        scratch_types=[
            pltpu.VMEM((BPW, 2), jnp.int32),
            pltpu.VMEM((NCHUNK, CHUNK), jnp.int32),
            pltpu.VMEM((NCHUNK, CHUNK), jnp.int32),
            pltpu.VMEM((BPW, D), jnp.float32),
            pltpu.VMEM((BPW, D), jnp.float32),
            pltpu.VMEM((BPW,), jnp.float32),
            pltpu.SemaphoreType.DMA,
        ],
    )
    return run(x.astype(jnp.int32), user_embedding, item_embedding)


# speedup vs baseline: 1.0038x; 1.0038x over previous
import jax
import jax.numpy as jnp
from jax import lax
from jax.experimental import pallas as pl
from jax.experimental.pallas import tpu as pltpu
from jax.experimental.pallas import tpu_sc as plsc

NC = 2
NS = 16
L = 16
NW = NC * NS

BATCH = 16384
D = 32
BPW = BATCH // NW
CHUNK = 128
NCHUNK = BPW // CHUNK
NGROUP = BPW // L


def _mf_body(uids_hbm, iids_hbm, uemb_hbm, iemb_hbm, out_hbm,
             uidx, iidx, urows, irows, outv, sem):
    wid = lax.axis_index("s") * NC + lax.axis_index("c")
    base = wid * BPW

    copies = []
    for c in range(NCHUNK):
        pltpu.sync_copy(uids_hbm.at[pl.ds(base + c * CHUNK, CHUNK)],
                        uidx.at[c])
        pltpu.sync_copy(iids_hbm.at[pl.ds(base + c * CHUNK, CHUNK)],
                        iidx.at[c])
        copies.append(pltpu.async_copy(
            uemb_hbm.at[uidx.at[c]],
            urows.at[pl.ds(c * CHUNK, CHUNK), :], sem))
        copies.append(pltpu.async_copy(
            iemb_hbm.at[iidx.at[c]],
            irows.at[pl.ds(c * CHUNK, CHUNK), :], sem))
    for cp in copies:
        cp.wait()

    iota = lax.broadcasted_iota(jnp.int32, (L,), 0)

    def group_body(g, _):
        rows = g * L + iota
        acc = jnp.zeros((L,), jnp.float32)
        for d in range(D):
            dvec = jnp.full((L,), d, jnp.int32)
            u = plsc.load_gather(urows, [rows, dvec])
            v = plsc.load_gather(irows, [rows, dvec])
            acc = acc + u * v
        outv[pl.ds(g * L, L)] = acc
        return 0

    lax.fori_loop(0, NGROUP, group_body, 0)

    pltpu.sync_copy(outv, out_hbm.at[pl.ds(base, BPW)])


@jax.jit
def kernel(x, user_embedding, item_embedding):
    uids = x[:, 0].astype(jnp.int32)
    iids = x[:, 1].astype(jnp.int32)
    mesh = plsc.VectorSubcoreMesh(core_axis_name="c", subcore_axis_name="s")
    run = pl.kernel(
        _mf_body,
        out_type=jax.ShapeDtypeStruct((BATCH,), jnp.float32),
        mesh=mesh,
        compiler_params=pltpu.CompilerParams(
            needs_layout_passes=False, use_tc_tiling_on_sc=False),
        scratch_types=[
            pltpu.VMEM((NCHUNK, CHUNK), jnp.int32),
            pltpu.VMEM((NCHUNK, CHUNK), jnp.int32),
            pltpu.VMEM((BPW, D), jnp.float32),
            pltpu.VMEM((BPW, D), jnp.float32),
            pltpu.VMEM((BPW,), jnp.float32),
            pltpu.SemaphoreType.DMA,
        ],
    )
    return run(uids, iids, user_embedding * 1.0, item_embedding * 1.0)


# speedup vs baseline: 1.0140x; 1.0102x over previous
import jax
import jax.numpy as jnp
from jax import lax
from jax.experimental import pallas as pl
from jax.experimental.pallas import tpu as pltpu
from jax.experimental.pallas import tpu_sc as plsc

NC = 2
NS = 16
L = 16
NW = NC * NS

BATCH = 16384
D = 32
BPW = BATCH // NW
CHUNK = 128
NCHUNK = BPW // CHUNK


def _mf_body(uids_hbm, iids_hbm, uemb_hbm, iemb_hbm, out_hbm,
             uidx, iidx, urows, irows, outv, sem):
    wid = lax.axis_index("s") * NC + lax.axis_index("c")

    pltpu.sync_copy(uids_hbm.at[wid], uidx)
    pltpu.sync_copy(iids_hbm.at[wid], iidx)

    copies = []
    for c in range(NCHUNK):
        copies.append(pltpu.async_copy(
            uemb_hbm.at[uidx.at[c]],
            urows.at[pl.ds(c * CHUNK, CHUNK), :], sem))
        copies.append(pltpu.async_copy(
            iemb_hbm.at[iidx.at[c]],
            irows.at[pl.ds(c * CHUNK, CHUNK), :], sem))
    for cp in copies:
        cp.wait()

    last_lane = lax.broadcasted_iota(jnp.int32, (L,), 0) == (L - 1)

    def row_body(r, _):
        acc = (urows[r, pl.ds(0, L)] * irows[r, pl.ds(0, L)]
               + urows[r, pl.ds(L, L)] * irows[r, pl.ds(L, L)])
        total = plsc.cumsum(acc)
        plsc.store_scatter(outv, [jnp.full((L,), r, jnp.int32)], total,
                           mask=last_lane)
        return 0

    lax.fori_loop(0, BPW, row_body, 0)

    pltpu.sync_copy(outv, out_hbm.at[pl.ds(wid * BPW, BPW)])


@jax.jit
def kernel(x, user_embedding, item_embedding):
    uids = x[:, 0].astype(jnp.int32).reshape(NW, NCHUNK, CHUNK)
    iids = x[:, 1].astype(jnp.int32).reshape(NW, NCHUNK, CHUNK)
    mesh = plsc.VectorSubcoreMesh(core_axis_name="c", subcore_axis_name="s")
    run = pl.kernel(
        _mf_body,
        out_type=jax.ShapeDtypeStruct((BATCH,), jnp.float32),
        mesh=mesh,
        compiler_params=pltpu.CompilerParams(
            needs_layout_passes=False, use_tc_tiling_on_sc=False),
        scratch_types=[
            pltpu.VMEM((NCHUNK, CHUNK), jnp.int32),
            pltpu.VMEM((NCHUNK, CHUNK), jnp.int32),
            pltpu.VMEM((BPW, D), jnp.float32),
            pltpu.VMEM((BPW, D), jnp.float32),
            pltpu.VMEM((BPW,), jnp.float32),
            pltpu.SemaphoreType.DMA,
        ],
    )
    return run(uids, iids, user_embedding, item_embedding)
